# baseline (speedup 1.0000x reference)
import functools

import jax
import jax.numpy as jnp
from jax import lax
from jax.experimental import pallas as pl
from jax.experimental.pallas import tpu as pltpu
from jax.experimental.pallas import tpu_sc as plsc

_NUM_FIELDS = 26
_VOCAB = 100000
_EMB_DIM = 16
_BATCH = 16384
_GROUP = 8

try:
    _info = plsc.get_sparse_core_info()
    _NC, _NS, _L = _info.num_cores, _info.num_subcores, _info.num_lanes
except Exception:
    _NC, _NS, _L = 2, 16, 16

_NW = _NC * _NS
_BPW = _BATCH // _NW
_SB = 4
_CB = _BPW // _SB
_GROUPS = _NUM_FIELDS * _VOCAB // _GROUP
_OUT_W = _NUM_FIELDS * _EMB_DIM


def _make_sc_gather():
    mesh = plsc.VectorSubcoreMesh(core_axis_name="c", subcore_axis_name="s")

    @functools.partial(
        pl.kernel,
        out_type=jax.ShapeDtypeStruct((_BATCH, _OUT_W), jnp.float32),
        mesh=mesh,
        compiler_params=pltpu.CompilerParams(
            use_tc_tiling_on_sc=True, needs_layout_passes=False),
        scratch_types=[
            pltpu.VMEM((_NUM_FIELDS, _CB), jnp.int32),
            pltpu.VMEM((_NUM_FIELDS, _CB), jnp.int32),
            pltpu.VMEM((_CB,), jnp.int32),
            pltpu.VMEM((_CB,), jnp.int32),
            pltpu.VMEM((_CB, _GROUP * _EMB_DIM), jnp.float32),
            pltpu.VMEM((_CB, _GROUP * _EMB_DIM), jnp.float32),
            pltpu.VMEM((_CB, _OUT_W), jnp.float32),
            pltpu.SemaphoreType.DMA,
            pltpu.SemaphoreType.DMA,
        ],
    )
    def k(*args):
        tab_refs = args[:_NUM_FIELDS]
        (g_hbm, r_hbm, out_hbm,
         gblk, rblk, gx0, gx1, gb0, gb1, ob, sem0, sem1) = args[_NUM_FIELDS:]
        wid = lax.axis_index("s") * _NC + lax.axis_index("c")
        gxs = (gx0, gx1)
        gbs = (gb0, gb1)
        sems = (sem0, sem1)

        def bounce(f):
            def body(j, _):
                s = pl.ds(j * _L, _L)
                gxs[f % 2][s] = gblk[f, s]
                return 0
            lax.fori_loop(0, _CB // _L, body, 0)

        def fire(f):
            return pltpu.async_copy(
                tab_refs[f].at[gxs[f % 2]], gbs[f % 2], sems[f % 2])

        def extract(f):
            gb = gbs[f % 2]

            def row_group(rg, _):
                i_vec = rg * _L + lax.iota(jnp.int32, _L)
                r_vec = rblk[f, pl.ds(rg * _L, _L)]
                col0 = r_vec * _EMB_DIM

                def dm_step(dm, _):
                    v = plsc.load_gather(gb, [i_vec, col0 + dm])
                    plsc.store_scatter(
                        ob, [i_vec, jnp.full_like(i_vec, f * _EMB_DIM + dm)],
                        v)
                    return 0
                lax.fori_loop(0, _EMB_DIM, dm_step, 0)
                return 0
            lax.fori_loop(0, _CB // _L, row_group, 0)

        for sb in range(_SB):
            pltpu.sync_copy(g_hbm.at[wid, sb], gblk)
            pltpu.sync_copy(r_hbm.at[wid, sb], rblk)
            descs = []
            for f in range(_NUM_FIELDS):
                bounce(f)
                descs.append(fire(f))
                if f > 0:
                    descs[f - 1].wait()
                    extract(f - 1)
            descs[_NUM_FIELDS - 1].wait()
            extract(_NUM_FIELDS - 1)
            b0 = wid * _BPW + sb * _CB
            pltpu.sync_copy(ob, out_hbm.at[pl.ds(b0, _CB)])

    return k


_sc_gather = _make_sc_gather()


def kernel(fixed_tensor, tables):
    idx = fixed_tensor.astype(jnp.int32)
    idx4 = idx.reshape(_NW, _SB, _CB, _NUM_FIELDS).transpose(0, 1, 3, 2)
    g4 = idx4 >> 3
    r4 = idx4 & 7
    tabs = [tables[f].reshape(_VOCAB // _GROUP, _GROUP * _EMB_DIM)
            for f in range(_NUM_FIELDS)]
    return _sc_gather(*tabs, g4, r4)


# speedup vs baseline: 1.6995x; 1.6995x over previous
import functools

import jax
import jax.numpy as jnp
from jax import lax
from jax.experimental import pallas as pl
from jax.experimental.pallas import tpu as pltpu
from jax.experimental.pallas import tpu_sc as plsc

_NUM_FIELDS = 26
_VOCAB = 100000
_EMB_DIM = 16
_BATCH = 16384

try:
    _info = plsc.get_sparse_core_info()
    _NC, _NS, _L = _info.num_cores, _info.num_subcores, _info.num_lanes
except Exception:
    _NC, _NS, _L = 2, 16, 16

_NW = _NC * _NS
_BPW = _BATCH // _NW


def _make_sc_gather():
    mesh = plsc.VectorSubcoreMesh(core_axis_name="c", subcore_axis_name="s")

    @functools.partial(
        pl.kernel,
        out_type=jax.ShapeDtypeStruct(
            (_BATCH, _NUM_FIELDS * _EMB_DIM), jnp.float32),
        mesh=mesh,
        compiler_params=pltpu.CompilerParams(use_tc_tiling_on_sc=False),
        scratch_types=(
            [pltpu.VMEM((_BPW,), jnp.int32) for _ in range(_NUM_FIELDS)]
            + [
                pltpu.VMEM((_BPW, _EMB_DIM), jnp.float32),
                pltpu.VMEM((_BPW, _EMB_DIM), jnp.float32),
                pltpu.SemaphoreType.DMA,
                pltpu.SemaphoreType.DMA,
                pltpu.SemaphoreType.DMA,
            ]
        ),
    )
    def k(tab_hbm, idx_hbm, out_hbm, *scratch):
        idx_vs = scratch[:_NUM_FIELDS]
        buf0, buf1, sem_idx, sem0, sem1 = scratch[_NUM_FIELDS:]
        wid = lax.axis_index("s") * _NC + lax.axis_index("c")
        b0 = wid * _BPW

        idx_descs = [
            pltpu.async_copy(
                idx_hbm.at[f, pl.ds(b0, _BPW)], idx_vs[f], sem_idx)
            for f in range(_NUM_FIELDS)
        ]

        bufs = (buf0, buf1)
        sems = (sem0, sem1)

        def fire(f):
            idx_descs[f].wait()
            return pltpu.async_copy(
                tab_hbm.at[f].at[idx_vs[f]], bufs[f % 2], sems[f % 2])

        def store(f):
            pltpu.sync_copy(
                bufs[f % 2],
                out_hbm.at[pl.ds(b0, _BPW), pl.ds(f * _EMB_DIM, _EMB_DIM)])

        descs = [fire(0)]
        for f in range(1, _NUM_FIELDS):
            descs.append(fire(f))
            descs[f - 1].wait()
            store(f - 1)
        descs[_NUM_FIELDS - 1].wait()
        store(_NUM_FIELDS - 1)

    return k


_sc_gather = _make_sc_gather()


def kernel(fixed_tensor, tables):
    idx_t = fixed_tensor.astype(jnp.int32).T
    return _sc_gather(tables, idx_t)
